# baseline (speedup 1.0000x reference)
import jax
import jax.numpy as jnp
from jax import lax
from jax.experimental import pallas as pl

N = 10000
E = 320000
D = 128
HID = 64
HEADS = 8
G = 16

NP = 10240
BN = 256
NBLK = NP // BN


def _elu(x):
    return jnp.where(x > 0.0, x, jnp.exp(jnp.minimum(x, 0.0)) - 1.0)


def _prep1_body(x_ref, w1_ref, wsc_ref, h_ref, ct_ref):
    h = jnp.dot(x_ref[...], w1_ref[...], preferred_element_type=jnp.float32)
    h_ref[...] = h
    ct_ref[...] = jnp.dot(h, wsc_ref[...], preferred_element_type=jnp.float32)


def _prep1(xp, W1, Wsc1):
    return pl.pallas_call(
        _prep1_body,
        grid=(NBLK,),
        in_specs=[
            pl.BlockSpec((BN, D), lambda i: (i, 0)),
            pl.BlockSpec((D, HEADS * HID), lambda i: (0, 0)),
            pl.BlockSpec((HEADS * HID, 32), lambda i: (0, 0)),
        ],
        out_specs=[
            pl.BlockSpec((BN, HEADS * HID), lambda i: (i, 0)),
            pl.BlockSpec((BN, 32), lambda i: (i, 0)),
        ],
        out_shape=[
            jax.ShapeDtypeStruct((NP, HEADS * HID), jnp.float32),
            jax.ShapeDtypeStruct((NP, 32), jnp.float32),
        ],
    )(xp, W1, Wsc1)


def _norm1_body(num_ref, den_ref, h1_ref, ct1_ref, b1_ref, w2_ref, wsc2_ref,
                h2_ref, ct2_ref):
    v = ct1_ref[:, 0:HEADS] + ct1_ref[:, 16:16 + HEADS]
    exself = jnp.exp(jnp.where(v >= 0.0, v, 0.2 * v))
    den = den_ref[...] + exself
    scl = jnp.concatenate(
        [jnp.broadcast_to(exself[:, k:k + 1], (BN, HID)) for k in range(HEADS)],
        axis=1)
    dnl = jnp.concatenate(
        [jnp.broadcast_to(den[:, k:k + 1], (BN, HID)) for k in range(HEADS)],
        axis=1)
    out1 = _elu((num_ref[...] + h1_ref[...] * scl) / dnl + b1_ref[...])
    h2 = jnp.dot(out1, w2_ref[...], preferred_element_type=jnp.float32)
    h2_ref[...] = h2
    ct2_ref[...] = jnp.dot(h2, wsc2_ref[...], preferred_element_type=jnp.float32)


def _norm1(num1, den1, h1, ct1, b1r, W2, Wsc2):
    return pl.pallas_call(
        _norm1_body,
        grid=(NBLK,),
        in_specs=[
            pl.BlockSpec((BN, HEADS * HID), lambda i: (i, 0)),
            pl.BlockSpec((BN, HEADS), lambda i: (i, 0)),
            pl.BlockSpec((BN, HEADS * HID), lambda i: (i, 0)),
            pl.BlockSpec((BN, 32), lambda i: (i, 0)),
            pl.BlockSpec((1, HEADS * HID), lambda i: (0, 0)),
            pl.BlockSpec((HEADS * HID, HID), lambda i: (0, 0)),
            pl.BlockSpec((HID, 32), lambda i: (0, 0)),
        ],
        out_specs=[
            pl.BlockSpec((BN, HID), lambda i: (i, 0)),
            pl.BlockSpec((BN, 32), lambda i: (i, 0)),
        ],
        out_shape=[
            jax.ShapeDtypeStruct((NP, HID), jnp.float32),
            jax.ShapeDtypeStruct((NP, 32), jnp.float32),
        ],
    )(num1, den1, h1, ct1, b1r, W2, Wsc2)


def _pool_body(num2_ref, den2_ref, h2_ref, ct2_ref, b2_ref, batch_ref,
               gsum_ref, gcnt_ref):
    i = pl.program_id(0)
    v = ct2_ref[:, 0:1] + ct2_ref[:, 16:17]
    exself = jnp.broadcast_to(jnp.exp(jnp.where(v >= 0.0, v, 0.2 * v)),
                              (BN, HID))
    den = den2_ref[...] + exself
    out2 = _elu((num2_ref[...] + h2_ref[...] * exself) / den + b2_ref[...])
    onehot = (batch_ref[...] ==
              lax.broadcasted_iota(jnp.int32, (BN, G), 1)).astype(jnp.float32)
    sums = lax.dot_general(onehot, out2, (((0,), (0,)), ((), ())),
                           preferred_element_type=jnp.float32)
    cnt = jnp.sum(onehot, axis=0, keepdims=True)

    @pl.when(i == 0)
    def _():
        gsum_ref[...] = sums
        gcnt_ref[...] = cnt

    @pl.when(i != 0)
    def _():
        gsum_ref[...] += sums
        gcnt_ref[...] += cnt


def _pool(num2, den2b, h2, ct2, b2r, batchp):
    return pl.pallas_call(
        _pool_body,
        grid=(NBLK,),
        in_specs=[
            pl.BlockSpec((BN, HID), lambda i: (i, 0)),
            pl.BlockSpec((BN, HID), lambda i: (i, 0)),
            pl.BlockSpec((BN, HID), lambda i: (i, 0)),
            pl.BlockSpec((BN, 32), lambda i: (i, 0)),
            pl.BlockSpec((1, HID), lambda i: (0, 0)),
            pl.BlockSpec((BN, 1), lambda i: (i, 0)),
        ],
        out_specs=[
            pl.BlockSpec((G, HID), lambda i: (0, 0)),
            pl.BlockSpec((1, G), lambda i: (0, 0)),
        ],
        out_shape=[
            jax.ShapeDtypeStruct((G, HID), jnp.float32),
            jax.ShapeDtypeStruct((1, G), jnp.float32),
        ],
    )(num2, den2b, h2, ct2, b2r, batchp)


def _head_body(gsum_ref, gcnt_ref, wc1_ref, bc1_ref, wc2_ref, bc2_ref,
               out_ref):
    gv = gsum_ref[...] / jnp.maximum(gcnt_ref[...], 1.0).T
    z = jnp.maximum(
        jnp.dot(gv, wc1_ref[...], preferred_element_type=jnp.float32)
        + bc1_ref[...], 0.0)
    out_ref[...] = jnp.dot(z, wc2_ref[...],
                           preferred_element_type=jnp.float32) + bc2_ref[...]


def _head(gsum, gcnt, Wc1, bc1r, Wc2, bc2r):
    return pl.pallas_call(
        _head_body,
        out_shape=jax.ShapeDtypeStruct((G, 1), jnp.float32),
    )(gsum, gcnt, Wc1, bc1r, Wc2, bc2r)


def kernel(x, edge_index, batch, W1, att_src1, att_dst1, b1, W2, att_src2,
           att_dst2, b2, Wc1, bc1, Wc2, bc2):
    xp = jnp.pad(x, ((0, NP - N), (0, 0)))
    src = edge_index[0]
    dst = edge_index[1]
    batchp = jnp.concatenate(
        [batch, jnp.full((NP - N,), G, batch.dtype)]).reshape(NP, 1)

    eye = jnp.eye(HEADS, dtype=jnp.float32)
    src8 = (att_src1.reshape(HEADS, HID, 1) *
            eye[:, None, :]).reshape(HEADS * HID, HEADS)
    dst8 = (att_dst1.reshape(HEADS, HID, 1) *
            eye[:, None, :]).reshape(HEADS * HID, HEADS)
    z8 = jnp.zeros((HEADS * HID, 8), jnp.float32)
    Wsc1 = jnp.concatenate([src8, z8, dst8, z8], axis=1)
    Wsc2 = jnp.concatenate(
        [att_src2.reshape(HID, 1), jnp.zeros((HID, 15), jnp.float32),
         att_dst2.reshape(HID, 1), jnp.zeros((HID, 15), jnp.float32)], axis=1)
    b1r = b1.reshape(1, HEADS * HID)
    b2r = b2.reshape(1, HID)
    bc1r = bc1.reshape(1, HID // 2)
    bc2r = bc2.reshape(1, 1)

    h1, ct1 = _prep1(xp, W1, Wsc1)
    e1 = ct1[src, 0:HEADS] + ct1[dst, 16:16 + HEADS]
    ex1 = jnp.exp(jnp.where(e1 >= 0.0, e1, 0.2 * e1))
    den1 = jax.ops.segment_sum(ex1, dst, num_segments=NP)
    msg1 = (h1[src].reshape(E, HEADS, HID) * ex1[:, :, None]
            ).reshape(E, HEADS * HID)
    num1 = jax.ops.segment_sum(msg1, dst, num_segments=NP)

    h2, ct2 = _norm1(num1, den1, h1, ct1, b1r, W2, Wsc2)

    e2 = ct2[src, 0:1] + ct2[dst, 16:17]
    ex2 = jnp.exp(jnp.where(e2 >= 0.0, e2, 0.2 * e2))
    den2 = jax.ops.segment_sum(ex2, dst, num_segments=NP)
    den2b = jnp.broadcast_to(den2, (NP, HID))
    num2 = jax.ops.segment_sum(h2[src] * ex2, dst, num_segments=NP)

    gsum, gcnt = _pool(num2, den2b, h2, ct2, b2r, batchp)
    return _head(gsum, gcnt, Wc1, bc1r, Wc2, bc2r)


# speedup vs baseline: 6.6838x; 6.6838x over previous
import jax
import jax.numpy as jnp
from jax import lax
from jax.experimental import pallas as pl

N = 10000
E = 320000
D = 128
HID = 64
HEADS = 8
G = 16

NP = 10240
BN = 256
NBLK = NP // BN


def _elu(x):
    return jnp.where(x > 0.0, x, jnp.exp(jnp.minimum(x, 0.0)) - 1.0)


def _prep1_body(x_ref, w1_ref, wsc_ref, h_ref, ct_ref):
    h = jnp.dot(x_ref[...], w1_ref[...], preferred_element_type=jnp.float32)
    h_ref[...] = h
    ct_ref[...] = jnp.dot(h, wsc_ref[...], preferred_element_type=jnp.float32)


def _prep1(xp, W1, Wsc1):
    return pl.pallas_call(
        _prep1_body,
        grid=(NBLK,),
        in_specs=[
            pl.BlockSpec((BN, D), lambda i: (i, 0)),
            pl.BlockSpec((D, HEADS * HID), lambda i: (0, 0)),
            pl.BlockSpec((HEADS * HID, 32), lambda i: (0, 0)),
        ],
        out_specs=[
            pl.BlockSpec((BN, HEADS * HID), lambda i: (i, 0)),
            pl.BlockSpec((BN, 32), lambda i: (i, 0)),
        ],
        out_shape=[
            jax.ShapeDtypeStruct((NP, HEADS * HID), jnp.float32),
            jax.ShapeDtypeStruct((NP, 32), jnp.float32),
        ],
    )(xp, W1, Wsc1)


def _norm1_body(num_ref, den_ref, h1_ref, ct1_ref, b1_ref, w2_ref, wsc2_ref,
                h2_ref, ct2_ref):
    v = ct1_ref[:, 0:HEADS] + ct1_ref[:, 16:16 + HEADS]
    exself = jnp.exp(jnp.where(v >= 0.0, v, 0.2 * v))
    den = den_ref[...] + exself
    scl = jnp.concatenate(
        [jnp.broadcast_to(exself[:, k:k + 1], (BN, HID)) for k in range(HEADS)],
        axis=1)
    dnl = jnp.concatenate(
        [jnp.broadcast_to(den[:, k:k + 1], (BN, HID)) for k in range(HEADS)],
        axis=1)
    out1 = _elu((num_ref[...] + h1_ref[...] * scl) / dnl + b1_ref[...])
    h2 = jnp.dot(out1, w2_ref[...], preferred_element_type=jnp.float32)
    h2_ref[...] = h2
    ct2_ref[...] = jnp.dot(h2, wsc2_ref[...], preferred_element_type=jnp.float32)


def _norm1(num1, den1, h1, ct1, b1r, W2, Wsc2):
    return pl.pallas_call(
        _norm1_body,
        grid=(NBLK,),
        in_specs=[
            pl.BlockSpec((BN, HEADS * HID), lambda i: (i, 0)),
            pl.BlockSpec((BN, HEADS), lambda i: (i, 0)),
            pl.BlockSpec((BN, HEADS * HID), lambda i: (i, 0)),
            pl.BlockSpec((BN, 32), lambda i: (i, 0)),
            pl.BlockSpec((1, HEADS * HID), lambda i: (0, 0)),
            pl.BlockSpec((HEADS * HID, HID), lambda i: (0, 0)),
            pl.BlockSpec((HID, 32), lambda i: (0, 0)),
        ],
        out_specs=[
            pl.BlockSpec((BN, HID), lambda i: (i, 0)),
            pl.BlockSpec((BN, 32), lambda i: (i, 0)),
        ],
        out_shape=[
            jax.ShapeDtypeStruct((NP, HID), jnp.float32),
            jax.ShapeDtypeStruct((NP, 32), jnp.float32),
        ],
    )(num1, den1, h1, ct1, b1r, W2, Wsc2)


def _pool_body(num2_ref, den2_ref, h2_ref, ct2_ref, b2_ref, batch_ref,
               gsum_ref, gcnt_ref):
    i = pl.program_id(0)
    v = ct2_ref[:, 0:1] + ct2_ref[:, 16:17]
    exself = jnp.broadcast_to(jnp.exp(jnp.where(v >= 0.0, v, 0.2 * v)),
                              (BN, HID))
    den = den2_ref[...] + exself
    out2 = _elu((num2_ref[...] + h2_ref[...] * exself) / den + b2_ref[...])
    onehot = (batch_ref[...] ==
              lax.broadcasted_iota(jnp.int32, (BN, G), 1)).astype(jnp.float32)
    sums = lax.dot_general(onehot, out2, (((0,), (0,)), ((), ())),
                           preferred_element_type=jnp.float32)
    cnt = jnp.sum(onehot, axis=0, keepdims=True)

    @pl.when(i == 0)
    def _():
        gsum_ref[...] = sums
        gcnt_ref[...] = cnt

    @pl.when(i != 0)
    def _():
        gsum_ref[...] += sums
        gcnt_ref[...] += cnt


def _pool(num2, den2b, h2, ct2, b2r, batchp):
    return pl.pallas_call(
        _pool_body,
        grid=(NBLK,),
        in_specs=[
            pl.BlockSpec((BN, HID), lambda i: (i, 0)),
            pl.BlockSpec((BN, HID), lambda i: (i, 0)),
            pl.BlockSpec((BN, HID), lambda i: (i, 0)),
            pl.BlockSpec((BN, 32), lambda i: (i, 0)),
            pl.BlockSpec((1, HID), lambda i: (0, 0)),
            pl.BlockSpec((BN, 1), lambda i: (i, 0)),
        ],
        out_specs=[
            pl.BlockSpec((G, HID), lambda i: (0, 0)),
            pl.BlockSpec((1, G), lambda i: (0, 0)),
        ],
        out_shape=[
            jax.ShapeDtypeStruct((G, HID), jnp.float32),
            jax.ShapeDtypeStruct((1, G), jnp.float32),
        ],
    )(num2, den2b, h2, ct2, b2r, batchp)


def _head_body(gsum_ref, gcnt_ref, wc1_ref, bc1_ref, wc2_ref, bc2_ref,
               out_ref):
    gv = gsum_ref[...] / jnp.maximum(gcnt_ref[...], 1.0).T
    z = jnp.maximum(
        jnp.dot(gv, wc1_ref[...], preferred_element_type=jnp.float32)
        + bc1_ref[...], 0.0)
    out_ref[...] = jnp.dot(z, wc2_ref[...],
                           preferred_element_type=jnp.float32) + bc2_ref[...]


def _head(gsum, gcnt, Wc1, bc1r, Wc2, bc2r):
    return pl.pallas_call(
        _head_body,
        out_shape=jax.ShapeDtypeStruct((G, 1), jnp.float32),
    )(gsum, gcnt, Wc1, bc1r, Wc2, bc2r)


def kernel(x, edge_index, batch, W1, att_src1, att_dst1, b1, W2, att_src2,
           att_dst2, b2, Wc1, bc1, Wc2, bc2):
    xp = jnp.pad(x, ((0, NP - N), (0, 0)))
    src = edge_index[0]
    dst = edge_index[1]
    batchp = jnp.concatenate(
        [batch, jnp.full((NP - N,), G, batch.dtype)]).reshape(NP, 1)

    eye = jnp.eye(HEADS, dtype=jnp.float32)
    src8 = (att_src1.reshape(HEADS, HID, 1) *
            eye[:, None, :]).reshape(HEADS * HID, HEADS)
    dst8 = (att_dst1.reshape(HEADS, HID, 1) *
            eye[:, None, :]).reshape(HEADS * HID, HEADS)
    z8 = jnp.zeros((HEADS * HID, 8), jnp.float32)
    Wsc1 = jnp.concatenate([src8, z8, dst8, z8], axis=1)
    Wsc2 = jnp.concatenate(
        [att_src2.reshape(HID, 1), jnp.zeros((HID, 15), jnp.float32),
         att_dst2.reshape(HID, 1), jnp.zeros((HID, 15), jnp.float32)], axis=1)
    b1r = b1.reshape(1, HEADS * HID)
    b2r = b2.reshape(1, HID)
    bc1r = bc1.reshape(1, HID // 2)
    bc2r = bc2.reshape(1, 1)

    h1, ct1 = _prep1(xp, W1, Wsc1)
    asrc1 = ct1[:N, 0:HEADS]
    adst1 = ct1[:N, 16:16 + HEADS]
    e1 = asrc1[src] + adst1[dst]
    ex1 = jnp.exp(jnp.where(e1 >= 0.0, e1, 0.2 * e1))
    den1 = jax.ops.segment_sum(ex1, dst, num_segments=N)
    h1n = h1[:N].reshape(N, HEADS, HID)
    msg1 = h1n[src] * ex1[:, :, None]
    num1 = jax.ops.segment_sum(msg1, dst, num_segments=N)
    num1p = jnp.pad(num1.reshape(N, HEADS * HID), ((0, NP - N), (0, 0)))
    den1p = jnp.pad(den1, ((0, NP - N), (0, 0)))

    h2, ct2 = _norm1(num1p, den1p, h1, ct1, b1r, W2, Wsc2)

    asrc2 = ct2[:N, 0:1]
    adst2 = ct2[:N, 16:17]
    e2 = asrc2[src] + adst2[dst]
    ex2 = jnp.exp(jnp.where(e2 >= 0.0, e2, 0.2 * e2))
    den2 = jax.ops.segment_sum(ex2, dst, num_segments=N)
    num2 = jax.ops.segment_sum(h2[:N][src] * ex2, dst, num_segments=N)
    num2p = jnp.pad(num2, ((0, NP - N), (0, 0)))
    den2b = jnp.broadcast_to(jnp.pad(den2, ((0, NP - N), (0, 0))), (NP, HID))

    gsum, gcnt = _pool(num2p, den2b, h2, ct2, b2r, batchp)
    return _head(gsum, gcnt, Wc1, bc1r, Wc2, bc2r)
